# Initial kernel scaffold; baseline (speedup 1.0000x reference)
import jax
import jax.numpy as jnp
from jax.experimental import pallas as pl


def kernel(x, ei0, ei1, ei2, W1_0, b1_0, W1_1, b1_1, W1_2, b1_2, W2_0, b2_0, W2_1, b2_1, W2_2, b2_2, W3_0, b3_0, W3_1, b3_1, W3_2, b3_2):
    raise NotImplementedError("write your pallas kernel here")


# speedup vs baseline: 3.0263x; 3.0263x over previous
import functools

import jax
import jax.numpy as jnp
from jax import lax
from jax.experimental import pallas as pl
from jax.experimental.pallas import tpu as pltpu
from jax.experimental.pallas import tpu_sc as plsc

_N = 10000
_E = 160000
_R = 3
_NTILES = 32
_CHUNK = 128
_S = 40
_PTILE = _S * _CHUNK
_EPAD = _NTILES * _PTILE
_NPAD = 10240
_ZROWS = _NPAD // 16

_mesh = plsc.VectorSubcoreMesh(core_axis_name="c", subcore_axis_name="s")
_sc_params = pltpu.CompilerParams(use_tc_tiling_on_sc=False)


@functools.partial(
    pl.kernel,
    out_type=jax.ShapeDtypeStruct((6, 2, _NPAD, 16), jnp.float32),
    mesh=_mesh,
    scratch_types=[
        pltpu.VMEM((6, _S, _CHUNK), jnp.int32),
        pltpu.VMEM((_CHUNK, 16), jnp.float32),
        pltpu.VMEM_SHARED((_NPAD, 16), jnp.float32),
    ],
    compiler_params=_sc_params,
)
def _sc_degrees(idx_hbm, ones_hbm, zeros_hbm, out_hbm, idx_v, ones_v, acc):
    c = lax.axis_index("c")
    s = lax.axis_index("s")
    w = c * 16 + s
    pltpu.sync_copy(idx_hbm.at[w], idx_v)
    pltpu.sync_copy(ones_hbm, ones_v)
    for j in range(6):
        pltpu.sync_copy(zeros_hbm.at[pl.ds(s * _ZROWS, _ZROWS)],
                        acc.at[pl.ds(s * _ZROWS, _ZROWS)])
        plsc.subcore_barrier()

        def step(i, carry):
            pltpu.sync_copy(ones_v, acc.at[idx_v.at[j, i]], add=True)
            return carry

        lax.fori_loop(0, _S, step, 0)
        plsc.subcore_barrier()
        pltpu.sync_copy(acc.at[pl.ds(s * _ZROWS, _ZROWS)],
                        out_hbm.at[j, c, pl.ds(s * _ZROWS, _ZROWS)])
        plsc.subcore_barrier()


def _make_sc_segsum(d):
    @functools.partial(
        pl.kernel,
        out_type=jax.ShapeDtypeStruct((_R, 2, _NPAD, d), jnp.float32),
        mesh=_mesh,
        scratch_types=[
            pltpu.VMEM((_R, _S, _CHUNK), jnp.int32),
            pltpu.VMEM((_R, _S, _CHUNK), jnp.int32),
            pltpu.VMEM((_CHUNK, d), jnp.float32),
            pltpu.SemaphoreType.DMA,
            pltpu.VMEM_SHARED((_NPAD, d), jnp.float32),
        ],
        compiler_params=_sc_params,
    )
    def segsum(tables, src_hbm, dst_hbm, zeros_hbm, out_hbm,
               src_v, dst_v, rows_v, sem, acc):
        c = lax.axis_index("c")
        s = lax.axis_index("s")
        w = c * 16 + s
        pltpu.sync_copy(src_hbm.at[w], src_v)
        pltpu.sync_copy(dst_hbm.at[w], dst_v)
        for r in range(_R):
            pltpu.sync_copy(zeros_hbm.at[pl.ds(s * _ZROWS, _ZROWS)],
                            acc.at[pl.ds(s * _ZROWS, _ZROWS)])
            plsc.subcore_barrier()

            def step(i, carry):
                pltpu.async_copy(tables.at[r].at[src_v.at[r, i]],
                                 rows_v, sem).wait()
                pltpu.sync_copy(rows_v, acc.at[dst_v.at[r, i]], add=True)
                return carry

            lax.fori_loop(0, _S, step, 0)
            plsc.subcore_barrier()
            pltpu.sync_copy(acc.at[pl.ds(s * _ZROWS, _ZROWS)],
                            out_hbm.at[r, c, pl.ds(s * _ZROWS, _ZROWS)])
            plsc.subcore_barrier()

    return segsum


_sc_segsum_128 = _make_sc_segsum(128)
_sc_segsum_64 = _make_sc_segsum(64)


_BN = 1000


def _norm_from(degref, j):
    deg = degref[j, 0, :, 0] + degref[j, 1, :, 0]
    return lax.rsqrt(jnp.maximum(deg, 1.0))


def _scale_matmul_body(h_ref, deg_ref, w_ref, out_ref):
    onorm = _norm_from(deg_ref, 0)
    out_ref[0] = jnp.dot(h_ref[...] * onorm[:, None], w_ref[0],
                         preferred_element_type=jnp.float32)


def _tc_scale_matmul(h, degp, w3):
    din, dout = w3.shape[1], w3.shape[2]
    grid = (_R, _N // _BN)
    return pl.pallas_call(
        _scale_matmul_body,
        grid=grid,
        in_specs=[
            pl.BlockSpec((_BN, din), lambda r, i: (i, 0)),
            pl.BlockSpec((1, 2, _BN, 16), lambda r, i: (2 * r, 0, i, 0)),
            pl.BlockSpec((1, din, dout), lambda r, i: (r, 0, 0)),
        ],
        out_specs=pl.BlockSpec((1, _BN, dout), lambda r, i: (r, i, 0)),
        out_shape=jax.ShapeDtypeStruct((_R, _N, dout), jnp.float32),
    )(h, degp, w3)


def _scale_only_body(h_ref, deg_ref, out_ref):
    onorm = _norm_from(deg_ref, 0)
    out_ref[0] = h_ref[...] * onorm[:, None]


def _tc_scale_only(h, degp):
    d = h.shape[1]
    grid = (_R, _N // _BN)
    return pl.pallas_call(
        _scale_only_body,
        grid=grid,
        in_specs=[
            pl.BlockSpec((_BN, d), lambda r, i: (i, 0)),
            pl.BlockSpec((1, 2, _BN, 16), lambda r, i: (2 * r, 0, i, 0)),
        ],
        out_specs=pl.BlockSpec((1, _BN, d), lambda r, i: (r, i, 0)),
        out_shape=jax.ShapeDtypeStruct((_R, _N, d), jnp.float32),
    )(h, degp)


def _combine_body(p_ref, deg_ref, b_ref, out_ref, *, relu):
    acc = None
    for r in range(_R):
        innorm = _norm_from(deg_ref, 2 * r + 1)
        part = (p_ref[r, 0] + p_ref[r, 1]) * innorm[:, None] + b_ref[r][None, :]
        acc = part if acc is None else acc + part
    out_ref[...] = jnp.maximum(acc, 0.0) if relu else acc


def _tc_combine(partials, degp, b, relu):
    d = partials.shape[-1]
    grid = (_N // _BN,)
    return pl.pallas_call(
        functools.partial(_combine_body, relu=relu),
        grid=grid,
        in_specs=[
            pl.BlockSpec((_R, 2, _BN, d), lambda i: (0, 0, i, 0)),
            pl.BlockSpec((6, 2, _BN, 16), lambda i: (0, 0, i, 0)),
            pl.BlockSpec((_R, d), lambda i: (0, 0)),
        ],
        out_specs=pl.BlockSpec((_BN, d), lambda i: (i, 0)),
        out_shape=jax.ShapeDtypeStruct((_N, d), jnp.float32),
    )(partials, degp, b)


def _final_body(p_ref, deg_ref, w_ref, b_ref, out_ref):
    acc = None
    for r in range(_R):
        innorm = _norm_from(deg_ref, 2 * r + 1)
        part = jnp.dot((p_ref[r, 0] + p_ref[r, 1]) * innorm[:, None], w_ref[r],
                       preferred_element_type=jnp.float32) + b_ref[r][None, :]
        acc = part if acc is None else acc + part
    out_ref[...] = acc


def _tc_final(partials, degp, w3p, b3p):
    grid = (_N // _BN,)
    return pl.pallas_call(
        _final_body,
        grid=grid,
        in_specs=[
            pl.BlockSpec((_R, 2, _BN, 64), lambda i: (0, 0, i, 0)),
            pl.BlockSpec((6, 2, _BN, 16), lambda i: (0, 0, i, 0)),
            pl.BlockSpec((_R, 64, 128), lambda i: (0, 0, 0)),
            pl.BlockSpec((_R, 128), lambda i: (0, 0)),
        ],
        out_specs=pl.BlockSpec((_BN, 128), lambda i: (i, 0)),
        out_shape=jax.ShapeDtypeStruct((_N, 128), jnp.float32),
    )(partials, degp, w3p, b3p)


def _pad_reshape(idx, fill):
    pad = jnp.full((_EPAD - _E,), fill, jnp.int32)
    return jnp.concatenate([idx, pad]).reshape(_NTILES, _S, _CHUNK)


def kernel(x, ei0, ei1, ei2, W1_0, b1_0, W1_1, b1_1, W1_2, b1_2,
           W2_0, b2_0, W2_1, b2_1, W2_2, b2_2,
           W3_0, b3_0, W3_1, b3_1, W3_2, b3_2):
    eis = [ei0, ei1, ei2]
    deg_idx = jnp.stack(
        [_pad_reshape(eis[r][i], _N) for r in range(_R) for i in (0, 1)],
        axis=1)
    src_idx = jnp.stack([_pad_reshape(eis[r][0], 0) for r in range(_R)], axis=1)
    dst_idx = jnp.stack([_pad_reshape(eis[r][1], _N) for r in range(_R)], axis=1)

    ones16 = jnp.ones((_CHUNK, 16), jnp.float32)
    zeros16 = jnp.zeros((_NPAD, 16), jnp.float32)
    zeros128 = jnp.zeros((_NPAD, 128), jnp.float32)
    zeros64 = jnp.zeros((_NPAD, 64), jnp.float32)

    degp = _sc_degrees(deg_idx, ones16, zeros16)

    w1 = jnp.stack([W1_0, W1_1, W1_2])
    b1 = jnp.stack([b1_0, b1_1, b1_2])
    w2 = jnp.stack([W2_0, W2_1, W2_2])
    b2 = jnp.stack([b2_0, b2_1, b2_2])
    w3 = jnp.stack([W3_0, W3_1, W3_2])
    b3 = jnp.stack([b3_0, b3_1, b3_2])
    w3p = jnp.pad(w3, ((0, 0), (0, 0), (0, 126)))
    b3p = jnp.pad(b3, ((0, 0), (0, 126)))

    hw = _tc_scale_matmul(x, degp, w1)
    parts = _sc_segsum_128(hw, src_idx, dst_idx, zeros128)
    h = _tc_combine(parts, degp, b1, relu=True)

    hw = _tc_scale_matmul(h, degp, w2)
    parts = _sc_segsum_64(hw, src_idx, dst_idx, zeros64)
    h = _tc_combine(parts, degp, b2, relu=True)

    hs = _tc_scale_only(h, degp)
    parts = _sc_segsum_64(hs, src_idx, dst_idx, zeros64)
    out = _tc_final(parts, degp, w3p, b3p)
    return out[:, :2]


# speedup vs baseline: 3.1844x; 1.0522x over previous
import functools

import jax
import jax.numpy as jnp
from jax import lax
from jax.experimental import pallas as pl
from jax.experimental.pallas import tpu as pltpu
from jax.experimental.pallas import tpu_sc as plsc

_N = 10000
_E = 160000
_R = 3
_NTILES = 32
_PTILE = 5120
_EPAD = _NTILES * _PTILE
_NPAD = 10240
_ZROWS = _NPAD // 16

_mesh = plsc.VectorSubcoreMesh(core_axis_name="c", subcore_axis_name="s")
_sc_params = pltpu.CompilerParams(use_tc_tiling_on_sc=False)


_DEG_K = 1
_DEG_S = _PTILE // (_DEG_K * 128)


@functools.partial(
    pl.kernel,
    out_type=jax.ShapeDtypeStruct((6, 2, _NPAD, 16), jnp.float32),
    mesh=_mesh,
    scratch_types=[
        pltpu.VMEM((6, _DEG_S, _DEG_K * 128), jnp.int32),
        pltpu.VMEM((_DEG_K * 128, 16), jnp.float32),
        pltpu.VMEM_SHARED((_NPAD, 16), jnp.float32),
    ],
    compiler_params=_sc_params,
)
def _sc_degrees(idx_hbm, ones_hbm, zeros_hbm, out_hbm, idx_v, ones_v, acc):
    c = lax.axis_index("c")
    s = lax.axis_index("s")
    w = c * 16 + s
    pltpu.sync_copy(idx_hbm.at[w], idx_v)
    pltpu.sync_copy(ones_hbm, ones_v)
    for j in range(6):
        pltpu.sync_copy(zeros_hbm.at[pl.ds(s * _ZROWS, _ZROWS)],
                        acc.at[pl.ds(s * _ZROWS, _ZROWS)])
        plsc.subcore_barrier()

        def step(i, carry):
            pltpu.sync_copy(ones_v, acc.at[idx_v.at[j, i]], add=True)
            return carry

        lax.fori_loop(0, _DEG_S, step, 0)
        plsc.subcore_barrier()
        pltpu.sync_copy(acc.at[pl.ds(s * _ZROWS, _ZROWS)],
                        out_hbm.at[j, c, pl.ds(s * _ZROWS, _ZROWS)])
        plsc.subcore_barrier()


def _make_sc_segsum(d, k, n_tab):
    steps = _PTILE // (k * 128)
    chunk = k * 128

    @functools.partial(
        pl.kernel,
        out_type=jax.ShapeDtypeStruct((n_tab, 2, _NPAD, d), jnp.float32),
        mesh=_mesh,
        scratch_types=[
            pltpu.VMEM((_R, steps, chunk), jnp.int32),
            pltpu.VMEM((_R, steps, chunk), jnp.int32),
            pltpu.VMEM((chunk, d), jnp.float32),
            pltpu.SemaphoreType.DMA,
            pltpu.VMEM_SHARED((_NPAD, d), jnp.float32),
        ],
        compiler_params=_sc_params,
    )
    def segsum(tables, src_hbm, dst_hbm, zeros_hbm, out_hbm,
               src_v, dst_v, rows_v, gsem, acc):
        c = lax.axis_index("c")
        s = lax.axis_index("s")
        w = c * 16 + s
        pltpu.sync_copy(src_hbm.at[w], src_v)
        pltpu.sync_copy(dst_hbm.at[w], dst_v)
        for t in range(n_tab):
            r = t % _R
            pltpu.sync_copy(zeros_hbm.at[pl.ds(s * _ZROWS, _ZROWS)],
                            acc.at[pl.ds(s * _ZROWS, _ZROWS)])
            plsc.subcore_barrier()

            def step(i, carry):
                pltpu.async_copy(tables.at[t].at[src_v.at[r, i]],
                                 rows_v, gsem).wait()
                pltpu.sync_copy(rows_v, acc.at[dst_v.at[r, i]],
                                add=True)
                return carry

            lax.fori_loop(0, steps, step, 0)
            plsc.subcore_barrier()
            pltpu.sync_copy(acc.at[pl.ds(s * _ZROWS, _ZROWS)],
                            out_hbm.at[t, c, pl.ds(s * _ZROWS, _ZROWS)])
            plsc.subcore_barrier()

    return segsum


_SEG_K = 4
_sc_segsum_64 = _make_sc_segsum(64, _SEG_K, _R)
_sc_segsum_64x2 = _make_sc_segsum(64, _SEG_K, 2 * _R)


_BN = 1000


def _norm_from(degref, j):
    deg = degref[j, 0, :, 0] + degref[j, 1, :, 0]
    return lax.rsqrt(jnp.maximum(deg, 1.0))


def _scale_matmul_body(h_ref, deg_ref, w_ref, out_ref):
    onorm = _norm_from(deg_ref, 0)
    out_ref[0] = jnp.dot(h_ref[...] * onorm[:, None], w_ref[0],
                         preferred_element_type=jnp.float32)


def _tc_scale_matmul(h, degp, w3):
    nr, din, dout = w3.shape
    grid = (nr, _N // _BN)
    return pl.pallas_call(
        _scale_matmul_body,
        grid=grid,
        in_specs=[
            pl.BlockSpec((_BN, din), lambda r, i: (i, 0)),
            pl.BlockSpec((1, 2, _BN, 16), lambda r, i: (2 * (r % _R), 0, i, 0)),
            pl.BlockSpec((1, din, dout), lambda r, i: (r, 0, 0)),
        ],
        out_specs=pl.BlockSpec((1, _BN, dout), lambda r, i: (r, i, 0)),
        out_shape=jax.ShapeDtypeStruct((nr, _N, dout), jnp.float32),
    )(h, degp, w3)


def _scale_only_body(h_ref, deg_ref, out_ref):
    onorm = _norm_from(deg_ref, 0)
    out_ref[0] = h_ref[...] * onorm[:, None]


def _tc_scale_only(h, degp):
    d = h.shape[1]
    grid = (_R, _N // _BN)
    return pl.pallas_call(
        _scale_only_body,
        grid=grid,
        in_specs=[
            pl.BlockSpec((_BN, d), lambda r, i: (i, 0)),
            pl.BlockSpec((1, 2, _BN, 16), lambda r, i: (2 * r, 0, i, 0)),
        ],
        out_specs=pl.BlockSpec((1, _BN, d), lambda r, i: (r, i, 0)),
        out_shape=jax.ShapeDtypeStruct((_R, _N, d), jnp.float32),
    )(h, degp)


def _combine_body(p_ref, deg_ref, b_ref, out_ref, *, relu):
    acc = None
    for r in range(_R):
        innorm = _norm_from(deg_ref, 2 * r + 1)
        part = (p_ref[r, 0] + p_ref[r, 1]) * innorm[:, None] + b_ref[r][None, :]
        acc = part if acc is None else acc + part
    out_ref[...] = jnp.maximum(acc, 0.0) if relu else acc


def _tc_combine(partials, degp, b, relu):
    d = partials.shape[-1]
    grid = (_N // _BN,)
    return pl.pallas_call(
        functools.partial(_combine_body, relu=relu),
        grid=grid,
        in_specs=[
            pl.BlockSpec((_R, 2, _BN, d), lambda i: (0, 0, i, 0)),
            pl.BlockSpec((6, 2, _BN, 16), lambda i: (0, 0, i, 0)),
            pl.BlockSpec((_R, d), lambda i: (0, 0)),
        ],
        out_specs=pl.BlockSpec((_BN, d), lambda i: (i, 0)),
        out_shape=jax.ShapeDtypeStruct((_N, d), jnp.float32),
    )(partials, degp, b)


def _combine2_body(p_ref, deg_ref, b_ref, out_ref):
    acc = None
    for r in range(_R):
        innorm = _norm_from(deg_ref, 2 * r + 1)
        part = jnp.concatenate(
            [p_ref[r, 0] + p_ref[r, 1], p_ref[r + _R, 0] + p_ref[r + _R, 1]],
            axis=1) * innorm[:, None] + b_ref[r][None, :]
        acc = part if acc is None else acc + part
    out_ref[...] = jnp.maximum(acc, 0.0)


def _tc_combine2(p, degp, b):
    grid = (_N // _BN,)
    return pl.pallas_call(
        _combine2_body,
        grid=grid,
        in_specs=[
            pl.BlockSpec((2 * _R, 2, _BN, 64), lambda i: (0, 0, i, 0)),
            pl.BlockSpec((6, 2, _BN, 16), lambda i: (0, 0, i, 0)),
            pl.BlockSpec((_R, 128), lambda i: (0, 0)),
        ],
        out_specs=pl.BlockSpec((_BN, 128), lambda i: (i, 0)),
        out_shape=jax.ShapeDtypeStruct((_N, 128), jnp.float32),
    )(p, degp, b)


def _final_body(p_ref, deg_ref, w_ref, b_ref, out_ref):
    acc = None
    for r in range(_R):
        innorm = _norm_from(deg_ref, 2 * r + 1)
        part = jnp.dot((p_ref[r, 0] + p_ref[r, 1]) * innorm[:, None], w_ref[r],
                       preferred_element_type=jnp.float32) + b_ref[r][None, :]
        acc = part if acc is None else acc + part
    out_ref[...] = acc


def _tc_final(partials, degp, w3p, b3p):
    grid = (_N // _BN,)
    return pl.pallas_call(
        _final_body,
        grid=grid,
        in_specs=[
            pl.BlockSpec((_R, 2, _BN, 64), lambda i: (0, 0, i, 0)),
            pl.BlockSpec((6, 2, _BN, 16), lambda i: (0, 0, i, 0)),
            pl.BlockSpec((_R, 64, 128), lambda i: (0, 0, 0)),
            pl.BlockSpec((_R, 128), lambda i: (0, 0)),
        ],
        out_specs=pl.BlockSpec((_BN, 128), lambda i: (i, 0)),
        out_shape=jax.ShapeDtypeStruct((_N, 128), jnp.float32),
    )(partials, degp, w3p, b3p)


def _pad_flat(idx, fill):
    pad = jnp.full((_EPAD - _E,), fill, jnp.int32)
    return jnp.concatenate([idx, pad]).reshape(_NTILES, _PTILE)


def _seg_shape(k):
    return (_NTILES, _R, _PTILE // (k * 128), k * 128)


def kernel(x, ei0, ei1, ei2, W1_0, b1_0, W1_1, b1_1, W1_2, b1_2,
           W2_0, b2_0, W2_1, b2_1, W2_2, b2_2,
           W3_0, b3_0, W3_1, b3_1, W3_2, b3_2):
    eis = [ei0, ei1, ei2]
    deg_idx = jnp.stack(
        [_pad_flat(eis[r][i], _N) for r in range(_R) for i in (0, 1)],
        axis=1).reshape(_NTILES, 6, _DEG_S, _DEG_K * 128)
    srcs = jnp.stack([_pad_flat(eis[r][0], 0) for r in range(_R)], axis=1)
    dsts = jnp.stack([_pad_flat(eis[r][1], _N) for r in range(_R)], axis=1)
    src4 = srcs.reshape(_seg_shape(_SEG_K))
    dst4 = dsts.reshape(_seg_shape(_SEG_K))

    ones16 = jnp.ones((_DEG_K * 128, 16), jnp.float32)
    zeros16 = jnp.zeros((_NPAD, 16), jnp.float32)
    zeros64 = jnp.zeros((_NPAD, 64), jnp.float32)

    degp = _sc_degrees(deg_idx, ones16, zeros16)

    w1 = jnp.stack([W1_0, W1_1, W1_2])
    b1 = jnp.stack([b1_0, b1_1, b1_2])
    w2 = jnp.stack([W2_0, W2_1, W2_2])
    b2 = jnp.stack([b2_0, b2_1, b2_2])
    w3 = jnp.stack([W3_0, W3_1, W3_2])
    b3 = jnp.stack([b3_0, b3_1, b3_2])
    w3p = jnp.pad(w3, ((0, 0), (0, 0), (0, 126)))
    b3p = jnp.pad(b3, ((0, 0), (0, 126)))

    w1s = jnp.concatenate([w1[:, :, :64], w1[:, :, 64:]], axis=0)
    hw6 = _tc_scale_matmul(x, degp, w1s)
    parts6 = _sc_segsum_64x2(hw6, src4, dst4, zeros64)
    h = _tc_combine2(parts6, degp, b1)

    hw = _tc_scale_matmul(h, degp, w2)
    parts = _sc_segsum_64(hw, src4, dst4, zeros64)
    h = _tc_combine(parts, degp, b2, relu=True)

    hs = _tc_scale_only(h, degp)
    parts = _sc_segsum_64(hs, src4, dst4, zeros64)
    out = _tc_final(parts, degp, w3p, b3p)
    return out[:, :2]
